# baseline (speedup 1.0000x reference)
import jax
import jax.numpy as jnp
from jax import lax
from jax.experimental import pallas as pl
from jax.experimental.pallas import tpu as pltpu
from jax.experimental.pallas import tpu_sc as plsc

_NW = 32
_S = 128
_TC = 512


def kernel(x, table):
    B, L = x.shape
    V, E = table.shape
    idx = x.astype(jnp.int32)
    tab_t = table.T

    grid_t = (V + _TC - 1) // _TC

    def _widen_body(t_ref, o_ref):
        o_ref[:, :E] = t_ref[...].T

    t128 = pl.pallas_call(
        _widen_body,
        grid=(grid_t,),
        in_specs=[pl.BlockSpec((E, _TC), lambda i: (0, i))],
        out_specs=pl.BlockSpec((_TC, 128), lambda i: (i, 0)),
        out_shape=jax.ShapeDtypeStruct((V, 128), table.dtype),
    )(tab_t)

    mesh = plsc.VectorSubcoreMesh(core_axis_name="core", subcore_axis_name="subcore")
    rows_per_worker = B // _NW
    chunks = rows_per_worker // 8
    lane_starts = list(range(0, L - 16, 16)) + [L - 16]

    @pl.kernel(
        out_type=jax.ShapeDtypeStruct((B, E, L), table.dtype),
        mesh=mesh,
        compiler_params=pltpu.CompilerParams(needs_layout_passes=False),
        scratch_types=[
            pltpu.VMEM((8, L), jnp.int32),
            pltpu.VMEM((L, 128), jnp.float32),
            pltpu.VMEM((E, L), jnp.float32),
        ],
    )
    def _gather(t_hbm, i_hbm, o_hbm, i_vmem, g_vmem, o_vmem):
        w = lax.axis_index("subcore") * 2 + lax.axis_index("core")
        b0 = w * rows_per_worker
        iota = lax.iota(jnp.int32, 16)

        @pl.loop(0, chunks)
        def _(c):
            cb = b0 + c * 8
            pltpu.sync_copy(i_hbm.at[pl.ds(cb, 8), :], i_vmem)
            for r in range(8):
                for lo in range(0, L, _S):
                    n = min(_S, L - lo)
                    pltpu.sync_copy(
                        t_hbm.at[i_vmem.at[r, pl.ds(lo, n)]],
                        g_vmem.at[pl.ds(lo, n), :],
                    )

                @pl.loop(0, E)
                def _(e):
                    for l0 in lane_starts:
                        vals = plsc.load_gather(
                            g_vmem, [l0 + iota, jnp.full((16,), e, jnp.int32)]
                        )
                        o_vmem[e, pl.ds(l0, 16)] = vals

                pltpu.sync_copy(o_vmem, o_hbm.at[cb + r])

    out_t = _gather(t128, idx)
    return jnp.swapaxes(out_t, 1, 2)


# speedup vs baseline: 2.6822x; 2.6822x over previous
import jax
import jax.numpy as jnp
from jax import lax
from jax.experimental import pallas as pl
from jax.experimental.pallas import tpu as pltpu
from jax.experimental.pallas import tpu_sc as plsc

_NW = 32
_S = 128
_TC = 2048


def kernel(x, table):
    B, L = x.shape
    V, E = table.shape
    idx = x.astype(jnp.int32)
    tab_t = table.T

    grid_t = (V + _TC - 1) // _TC

    def _widen_body(t_ref, o_ref):
        o_ref[:, :E] = t_ref[...].T

    t128 = pl.pallas_call(
        _widen_body,
        grid=(grid_t,),
        in_specs=[pl.BlockSpec((E, _TC), lambda i: (0, i))],
        out_specs=pl.BlockSpec((_TC, 128), lambda i: (i, 0)),
        out_shape=jax.ShapeDtypeStruct((V, 128), table.dtype),
        compiler_params=pltpu.CompilerParams(dimension_semantics=("parallel",)),
    )(tab_t)

    mesh = plsc.VectorSubcoreMesh(core_axis_name="core", subcore_axis_name="subcore")
    rows_per_worker = B // _NW
    chunks = rows_per_worker // 8

    @pl.kernel(
        out_type=jax.ShapeDtypeStruct((B, L, 128), table.dtype),
        mesh=mesh,
        scratch_types=[
            pltpu.VMEM((8, L), jnp.int32),
            pltpu.VMEM((L, 128), jnp.float32),
        ],
    )
    def _gather(t_hbm, i_hbm, o_hbm, i_vmem, g_vmem):
        w = lax.axis_index("subcore") * 2 + lax.axis_index("core")
        b0 = w * rows_per_worker

        @pl.loop(0, chunks)
        def _(c):
            cb = b0 + c * 8
            pltpu.sync_copy(i_hbm.at[pl.ds(cb, 8), :], i_vmem)
            for r in range(8):
                for lo in range(0, L, _S):
                    n = min(_S, L - lo)
                    pltpu.sync_copy(
                        t_hbm.at[i_vmem.at[r, pl.ds(lo, n)]],
                        g_vmem.at[pl.ds(lo, n), :],
                    )
                pltpu.sync_copy(g_vmem, o_hbm.at[cb + r])

    return _gather(t128, idx)[:, :, :E]


# speedup vs baseline: 3.2198x; 1.2004x over previous
import jax
import jax.numpy as jnp
from jax import lax
from jax.experimental import pallas as pl
from jax.experimental.pallas import tpu as pltpu
from jax.experimental.pallas import tpu_sc as plsc

_NW = 32
_S = 128
_TC = 2048


def kernel(x, table):
    B, L = x.shape
    V, E = table.shape
    idx = x.astype(jnp.int32)
    tab_t = table.T

    grid_t = (V + _TC - 1) // _TC

    def _widen_body(t_ref, o_ref):
        o_ref[:, :E] = t_ref[...].T

    t128 = pl.pallas_call(
        _widen_body,
        grid=(grid_t,),
        in_specs=[pl.BlockSpec((E, _TC), lambda i: (0, i))],
        out_specs=pl.BlockSpec((_TC, 128), lambda i: (i, 0)),
        out_shape=jax.ShapeDtypeStruct((V, 128), table.dtype),
        compiler_params=pltpu.CompilerParams(dimension_semantics=("parallel",)),
    )(tab_t)

    mesh = plsc.VectorSubcoreMesh(core_axis_name="core", subcore_axis_name="subcore")
    rows_per_worker = B // _NW
    chunks = rows_per_worker // 8

    @pl.kernel(
        out_type=jax.ShapeDtypeStruct((B, L, 128), table.dtype),
        mesh=mesh,
        scratch_types=[
            pltpu.VMEM((8, L), jnp.int32),
            pltpu.VMEM((2, L, 128), jnp.float32),
            pltpu.SemaphoreType.DMA,
            pltpu.SemaphoreType.DMA,
            pltpu.SemaphoreType.DMA,
        ],
    )
    def _gather(t_hbm, i_hbm, o_hbm, i_vmem, g_vmem, sem_g0, sem_g1, sem_o):
        w = lax.axis_index("subcore") * 2 + lax.axis_index("core")
        b0 = w * rows_per_worker
        sems = (sem_g0, sem_g1)

        def fire_gathers(r, cb):
            for lo in range(0, L, _S):
                n = min(_S, L - lo)
                pltpu.async_copy(
                    t_hbm.at[i_vmem.at[r, pl.ds(lo, n)]],
                    g_vmem.at[r % 2, pl.ds(lo, n), :],
                    sems[r % 2],
                )

        def wait_gathers(r):
            for lo in range(0, L, _S):
                n = min(_S, L - lo)
                pltpu.make_async_copy(
                    t_hbm.at[i_vmem.at[r, pl.ds(lo, n)]],
                    g_vmem.at[r % 2, pl.ds(lo, n), :],
                    sems[r % 2],
                ).wait()

        def fire_out(r, cb):
            pltpu.async_copy(g_vmem.at[r % 2], o_hbm.at[cb + r], sem_o)

        def wait_out(r, cb):
            pltpu.make_async_copy(
                g_vmem.at[r % 2], o_hbm.at[cb + r], sem_o
            ).wait()

        @pl.loop(0, chunks)
        def _(c):
            cb = b0 + c * 8
            pltpu.sync_copy(i_hbm.at[pl.ds(cb, 8), :], i_vmem)
            fire_gathers(0, cb)
            for r in range(1, 8):
                if r >= 2:
                    wait_out(r - 2, cb)
                fire_gathers(r, cb)
                wait_gathers(r - 1)
                fire_out(r - 1, cb)
            wait_gathers(7)
            fire_out(7, cb)
            wait_out(6, cb)
            wait_out(7, cb)

    return _gather(t128, idx)[:, :, :E]
